# baseline (speedup 1.0000x reference)
import jax
import jax.numpy as jnp
from jax.experimental import pallas as pl
from jax.experimental.pallas import tpu as pltpu

_LANES = 128
_BB = 256


def _crf_block_kernel(feats2d_ref, transB_ref, transT_ref, score_ref, path_ref,
                      hist_ref, tag_ref):
    S = transT_ref.shape[0]
    BB = feats2d_ref.shape[0]
    T = feats2d_ref.shape[1] // S
    L = transB_ref.shape[2]
    g = pl.program_id(0)
    nb = pl.num_programs(0) - 1
    slot = jax.lax.rem(g, 2)
    pslot = jax.lax.rem(g + 1, 2)

    sidx = jax.lax.broadcasted_iota(jnp.int32, (S, BB), 0).astype(jnp.float32)
    transT = transT_ref[...]

    def bwd_step(k, tag):
        onehot = jnp.where(sidx == tag, 1.0, 0.0)
        trans_row = jax.lax.dot_general(
            transT, onehot,
            dimension_numbers=(((1,), (0,)), ((), ())),
            preferred_element_type=jnp.float32,
            precision=jax.lax.Precision.HIGHEST,
        )
        cand = trans_row + hist_ref[pslot * T + (T - k - 1)]
        mb = jnp.max(cand, axis=0, keepdims=True)
        cur = jnp.min(
            jnp.where(cand == mb, sidx, float(S)), axis=0, keepdims=True
        )
        path_ref[T - k - 1] = cur.astype(jnp.int32)
        return cur

    def fwd_step(dA, dB, feat_t, t_idx):
        mA = mB = None
        for h in range(2):
            aA = aB = None
            for jj in range(S // 2):
                j = h * (S // 2) + jj
                row = transB_ref[j]
                cA = row + dA[j:j + 1, :]
                cB = row + dB[j:j + 1, :]
                aA = cA if aA is None else jnp.maximum(aA, cA)
                aB = cB if aB is None else jnp.maximum(aB, cB)
            mA = aA if mA is None else jnp.maximum(mA, aA)
            mB = aB if mB is None else jnp.maximum(mB, aB)
        ndA = mA + feat_t[:, :L]
        ndB = mB + feat_t[:, L:]
        hist_ref[slot * T + t_idx, :, :L] = ndA
        hist_ref[slot * T + t_idx, :, L:] = ndB
        return ndA, ndB

    @pl.when(g > 0)
    def _prev_tag_row():
        path_ref[T - 1] = tag_ref[pslot].astype(jnp.int32)

    @pl.when(g < nb)
    def _fwd_and_bwd():
        hist_ref[slot * T] = jnp.full((S, BB), -10000.0, dtype=jnp.float32)
        tag0 = tag_ref[pslot]

        def body(k2, carry):
            dA, dB, tag = carry
            c0 = feats2d_ref[:, pl.ds(k2 * (2 * S), 2 * S)]
            f1 = jnp.transpose(c0[:, S:])
            dA, dB = fwd_step(dA, dB, f1, 2 * k2 + 1)
            tag = bwd_step(2 * k2 + 1, tag)
            c1 = feats2d_ref[:, pl.ds((k2 + 1) * (2 * S), 2 * S)]
            f2 = jnp.transpose(c1[:, :S])
            dA, dB = fwd_step(dA, dB, f2, 2 * k2 + 2)
            tag = bwd_step(2 * k2 + 2, tag)
            return dA, dB, tag

        d0 = jnp.full((S, L), -10000.0, dtype=jnp.float32)
        dA, dB, tag = jax.lax.fori_loop(
            0, (T - 2) // 2, body, (d0, d0, tag0))

        cL = feats2d_ref[:, T * S - 2 * S:]
        fL = jnp.transpose(cL[:, S:])
        fA, fB = fwd_step(dA, dB, fL, T - 1)
        bwd_step(T - 1, tag)

        final_delta = jnp.concatenate([fA, fB], axis=1)
        m2 = jnp.max(final_delta, axis=0, keepdims=True)
        score_ref[0] = m2
        last_tag = jnp.min(
            jnp.where(final_delta == m2, sidx, float(S)), axis=0, keepdims=True
        )
        tag_ref[slot] = last_tag

    @pl.when(g == nb)
    def _bwd_only():
        jax.lax.fori_loop(1, T, bwd_step, tag_ref[pslot], unroll=2)


def kernel(feats, transitions):
    B, T, S = feats.shape
    bb = _BB
    nb = B // bb

    feats2d = feats.reshape(B, T * S)
    transB = jnp.broadcast_to(transitions.T[:, :, None], (S, S, _LANES))
    transT = transitions.T

    score, pathT = pl.pallas_call(
        _crf_block_kernel,
        grid=(nb + 1,),
        in_specs=[
            pl.BlockSpec((bb, T * S), lambda b: (jnp.minimum(b, nb - 1), 0)),
            pl.BlockSpec((S, S, _LANES), lambda b: (0, 0, 0)),
            pl.BlockSpec((S, S), lambda b: (0, 0)),
        ],
        out_specs=[
            pl.BlockSpec((1, 1, bb), lambda b: (0, 0, jnp.minimum(b, nb - 1))),
            pl.BlockSpec((T, 1, bb), lambda b: (0, 0, jnp.maximum(b - 1, 0))),
        ],
        out_shape=[
            jax.ShapeDtypeStruct((1, 1, B), jnp.float32),
            jax.ShapeDtypeStruct((T, 1, B), jnp.int32),
        ],
        scratch_shapes=[
            pltpu.VMEM((2 * T, S, bb), jnp.float32),
            pltpu.VMEM((2, 1, bb), jnp.float32),
        ],
        compiler_params=pltpu.CompilerParams(
            dimension_semantics=("arbitrary",),
        ),
    )(feats2d, transB, transT)

    return score.reshape(B), pathT.reshape(T, B).T


# speedup vs baseline: 1.7500x; 1.7500x over previous
import jax
import jax.numpy as jnp
from jax.experimental import pallas as pl
from jax.experimental.pallas import tpu as pltpu

_LANES = 128
_BB = 256


def _crf_block_kernel(featsT_ref, transB_ref, transT_ref, score_ref, path_ref,
                      hist_ref, tag_ref):
    T, S, BB = featsT_ref.shape
    L = transB_ref.shape[2]
    g = pl.program_id(0)
    nb = pl.num_programs(0) - 1
    slot = jax.lax.rem(g, 2)
    pslot = jax.lax.rem(g + 1, 2)

    sidx = jax.lax.broadcasted_iota(jnp.int32, (S, BB), 0).astype(jnp.float32)
    transT = transT_ref[...]

    def bwd_step(k, tag):
        onehot = jnp.where(sidx == tag, 1.0, 0.0)
        trans_row = jax.lax.dot_general(
            transT, onehot,
            dimension_numbers=(((1,), (0,)), ((), ())),
            preferred_element_type=jnp.float32,
            precision=jax.lax.Precision.HIGHEST,
        )
        cand = trans_row + hist_ref[pslot * T + (T - k - 1)]
        mb = jnp.max(cand, axis=0, keepdims=True)
        cur = jnp.min(
            jnp.where(cand == mb, sidx, float(S)), axis=0, keepdims=True
        )
        path_ref[T - k - 1] = cur.astype(jnp.int32)
        return cur

    @pl.when(g > 0)
    def _prev_tag_row():
        path_ref[T - 1] = tag_ref[pslot].astype(jnp.int32)

    @pl.when(g < nb)
    def _fwd_and_bwd():
        hist_ref[slot * T] = jnp.full((S, BB), -10000.0, dtype=jnp.float32)
        tag0 = tag_ref[pslot]

        def body(k, carry):
            dA, dB, tag = carry
            feat_t = featsT_ref[k]
            mA = mB = None
            for h in range(2):
                aA = aB = None
                for jj in range(S // 2):
                    j = h * (S // 2) + jj
                    row = transB_ref[j]
                    cA = row + dA[j:j + 1, :]
                    cB = row + dB[j:j + 1, :]
                    aA = cA if aA is None else jnp.maximum(aA, cA)
                    aB = cB if aB is None else jnp.maximum(aB, cB)
                mA = aA if mA is None else jnp.maximum(mA, aA)
                mB = aB if mB is None else jnp.maximum(mB, aB)
            ndA = mA + feat_t[:, :L]
            ndB = mB + feat_t[:, L:]
            hist_ref[slot * T + k, :, :L] = ndA
            hist_ref[slot * T + k, :, L:] = ndB
            cur = bwd_step(k, tag)
            return ndA, ndB, cur

        d0 = jnp.full((S, L), -10000.0, dtype=jnp.float32)
        fA, fB, _ = jax.lax.fori_loop(1, T, body, (d0, d0, tag0), unroll=2)

        final_delta = jnp.concatenate([fA, fB], axis=1)
        m2 = jnp.max(final_delta, axis=0, keepdims=True)
        score_ref[0] = m2
        last_tag = jnp.min(
            jnp.where(final_delta == m2, sidx, float(S)), axis=0, keepdims=True
        )
        tag_ref[slot] = last_tag

    @pl.when(g == nb)
    def _bwd_only():
        jax.lax.fori_loop(1, T, bwd_step, tag_ref[pslot], unroll=2)


def kernel(feats, transitions):
    B, T, S = feats.shape
    bb = _BB
    nb = B // bb

    featsT = jnp.transpose(feats, (1, 2, 0))
    transB = jnp.broadcast_to(transitions.T[:, :, None], (S, S, _LANES))
    transT = transitions.T

    score, pathT = pl.pallas_call(
        _crf_block_kernel,
        grid=(nb + 1,),
        in_specs=[
            pl.BlockSpec((T, S, bb), lambda b: (0, 0, jnp.minimum(b, nb - 1))),
            pl.BlockSpec((S, S, _LANES), lambda b: (0, 0, 0)),
            pl.BlockSpec((S, S), lambda b: (0, 0)),
        ],
        out_specs=[
            pl.BlockSpec((1, 1, bb), lambda b: (0, 0, jnp.minimum(b, nb - 1))),
            pl.BlockSpec((T, 1, bb), lambda b: (0, 0, jnp.maximum(b - 1, 0))),
        ],
        out_shape=[
            jax.ShapeDtypeStruct((1, 1, B), jnp.float32),
            jax.ShapeDtypeStruct((T, 1, B), jnp.int32),
        ],
        scratch_shapes=[
            pltpu.VMEM((2 * T, S, bb), jnp.float32),
            pltpu.VMEM((2, 1, bb), jnp.float32),
        ],
        compiler_params=pltpu.CompilerParams(
            dimension_semantics=("arbitrary",),
        ),
    )(featsT, transB, transT)

    return score.reshape(B), pathT.reshape(T, B).T
